# baseline (speedup 1.0000x reference)
import functools

import jax
import jax.numpy as jnp
from jax import lax
from jax.experimental import pallas as pl
from jax.experimental.pallas import tpu as pltpu
from jax.experimental.pallas import tpu_sc as plsc

_NW = 32
_W = 256


@functools.lru_cache(maxsize=None)
def _build_gather(n_idx: int, vocab: int, emb: int):
    per_w = n_idx // _NW
    steps = per_w // _W
    gps = _W // 8
    mesh = plsc.VectorSubcoreMesh(core_axis_name="core", subcore_axis_name="subcore")

    @functools.partial(
        pl.kernel,
        out_type=jax.ShapeDtypeStruct((n_idx, emb), jnp.float32),
        mesh=mesh,
        compiler_params=pltpu.CompilerParams(needs_layout_passes=False),
        scratch_types=[
            pltpu.VMEM((vocab * emb,), jnp.float32),
            pltpu.VMEM((_W,), jnp.int32),
            pltpu.VMEM((_W,), jnp.int32),
            pltpu.VMEM((_W, emb), jnp.float32),
            pltpu.VMEM((_W, emb), jnp.float32),
            pltpu.SemaphoreType.DMA,
            pltpu.SemaphoreType.DMA,
            pltpu.SemaphoreType.DMA,
            pltpu.SemaphoreType.DMA,
            pltpu.SemaphoreType.DMA,
        ],
    )
    def gather_kernel(table_hbm, idx_hbm, out_hbm, tab_v, ibuf0, ibuf1,
                      obuf0, obuf1, tsem, isem0, isem1, osem0, osem1):
        wid = lax.axis_index("subcore") * 2 + lax.axis_index("core")
        ibase = wid * per_w

        pltpu.make_async_copy(table_hbm, tab_v, tsem).start()
        pltpu.make_async_copy(table_hbm, tab_v, tsem).wait()

        ibufs = (ibuf0, ibuf1)
        obufs = (obuf0, obuf1)
        isems = (isem0, isem1)
        osems = (osem0, osem1)

        def idx_copy(s, b):
            return pltpu.make_async_copy(
                idx_hbm.at[pl.ds(ibase + s * _W, _W)], ibufs[b], isems[b])

        def out_copy(s, b):
            return pltpu.make_async_copy(
                obufs[b],
                out_hbm.at[pl.ds(ibase + s * _W, _W)],
                osems[b])

        idx_copy(0, 0).start()
        idx_copy(1, 1).start()

        iota = lax.iota(jnp.int32, 16)

        def half(s, b):
            idx_copy(s, b).wait()

            @pl.when(s >= 2)
            def _():
                out_copy(s - 2, b).wait()

            @pl.loop(0, _W // 16)
            def _(i):
                enc = ibufs[b][pl.ds(i * 16, 16)]
                for l in range(16):
                    base = jnp.broadcast_to(enc[l], (16,)) * emb + iota
                    for k in range(emb // 16):
                        v = plsc.load_gather(tab_v, [base + k * 16])
                        obufs[b][i * 16 + l, pl.ds(k * 16, 16)] = v

            out_copy(s, b).start()

            @pl.when(s + 2 < steps)
            def _():
                idx_copy(s + 2, b).start()

        @pl.loop(0, steps, step=2)
        def _(s):
            half(s, 0)
            half(s + 1, 1)

        out_copy(steps - 2, 0).wait()
        out_copy(steps - 1, 1).wait()

    return gather_kernel


def _mask_body(m_ref, o_ref):
    o_ref[...] = m_ref[...]


@functools.lru_cache(maxsize=None)
def _build_mask_copy(n: int, c: int, dtype_name: str):
    blk = 8192
    dtype = jnp.dtype(dtype_name)
    return pl.pallas_call(
        _mask_body,
        grid=(n // blk,),
        in_specs=[pl.BlockSpec((blk, c), lambda i: (i, 0))],
        out_specs=pl.BlockSpec((blk, c), lambda i: (i, 0)),
        out_shape=jax.ShapeDtypeStruct((n, c), dtype),
    )


def kernel(encodings, mask, table):
    n_tok, chr_len = encodings.shape
    vocab, emb = table.shape
    n_idx = n_tok * chr_len
    idx = encodings.reshape(n_idx).astype(jnp.int32)
    out = _build_gather(n_idx, vocab, emb)(table.reshape(vocab * emb), idx)
    mask_out = _build_mask_copy(n_tok, chr_len, mask.dtype.name)(mask)
    return out.reshape(n_tok, chr_len, emb), mask_out


# speedup vs baseline: 1.6334x; 1.6334x over previous
import functools

import jax
import jax.numpy as jnp
from jax import lax
from jax.experimental import pallas as pl
from jax.experimental.pallas import tpu as pltpu
from jax.experimental.pallas import tpu_sc as plsc

_NW = 32
_W = 256


@functools.lru_cache(maxsize=None)
def _build_gather(n_idx: int, vocab: int, emb: int):
    per_w = n_idx // _NW
    steps = per_w // _W
    gps = _W // 8
    mesh = plsc.VectorSubcoreMesh(core_axis_name="core", subcore_axis_name="subcore")

    @functools.partial(
        pl.kernel,
        out_type=jax.ShapeDtypeStruct((n_idx, emb), jnp.float32),
        mesh=mesh,
        compiler_params=pltpu.CompilerParams(needs_layout_passes=False),
        scratch_types=[
            pltpu.VMEM((vocab * emb,), jnp.float32),
            pltpu.VMEM((_W,), jnp.int32),
            pltpu.VMEM((_W,), jnp.int32),
            pltpu.VMEM((_W, emb), jnp.float32),
            pltpu.VMEM((_W, emb), jnp.float32),
            pltpu.SemaphoreType.DMA,
            pltpu.SemaphoreType.DMA,
            pltpu.SemaphoreType.DMA,
            pltpu.SemaphoreType.DMA,
            pltpu.SemaphoreType.DMA,
        ],
    )
    def gather_kernel(table_hbm, idx_hbm, out_hbm, tab_v, ibuf0, ibuf1,
                      obuf0, obuf1, tsem, isem0, isem1, osem0, osem1):
        wid = lax.axis_index("subcore") * 2 + lax.axis_index("core")
        ibase = wid * per_w

        pltpu.make_async_copy(table_hbm, tab_v, tsem).start()
        pltpu.make_async_copy(table_hbm, tab_v, tsem).wait()

        ibufs = (ibuf0, ibuf1)
        obufs = (obuf0, obuf1)
        isems = (isem0, isem1)
        osems = (osem0, osem1)

        def idx_copy(s, b):
            return pltpu.make_async_copy(
                idx_hbm.at[pl.ds(ibase + s * _W, _W)], ibufs[b], isems[b])

        def out_copy(s, b):
            return pltpu.make_async_copy(
                obufs[b],
                out_hbm.at[pl.ds(ibase + s * _W, _W)],
                osems[b])

        idx_copy(0, 0).start()
        idx_copy(1, 1).start()

        iota = lax.iota(jnp.int32, 16)

        def half(s, b):
            idx_copy(s, b).wait()

            @pl.when(s >= 2)
            def _():
                out_copy(s - 2, b).wait()

            @pl.loop(0, _W // 16)
            def _(i):
                enc = ibufs[b][pl.ds(i * 16, 16)]
                for l0 in range(0, 16, 2):
                    vals = []
                    for l in (l0, l0 + 1):
                        base = jnp.broadcast_to(enc[l], (16,)) * emb + iota
                        for k in range(emb // 16):
                            vals.append(plsc.load_gather(tab_v, [base + k * 16]))
                    for dl in (0, 1):
                        for k in range(emb // 16):
                            obufs[b][i * 16 + l0 + dl, pl.ds(k * 16, 16)] = (
                                vals[dl * (emb // 16) + k])

            out_copy(s, b).start()

            @pl.when(s + 2 < steps)
            def _():
                idx_copy(s + 2, b).start()

        @pl.loop(0, steps, step=2)
        def _(s):
            half(s, 0)
            half(s + 1, 1)

        out_copy(steps - 2, 0).wait()
        out_copy(steps - 1, 1).wait()

    return gather_kernel


def _mask_body(m_ref, o_ref):
    o_ref[...] = m_ref[...]


@functools.lru_cache(maxsize=None)
def _build_mask_copy(n: int, c: int, dtype_name: str):
    blk = 8192
    dtype = jnp.dtype(dtype_name)
    return pl.pallas_call(
        _mask_body,
        grid=(n // blk,),
        in_specs=[pl.BlockSpec((blk, c), lambda i: (i, 0))],
        out_specs=pl.BlockSpec((blk, c), lambda i: (i, 0)),
        out_shape=jax.ShapeDtypeStruct((n, c), dtype),
    )


def kernel(encodings, mask, table):
    n_tok, chr_len = encodings.shape
    vocab, emb = table.shape
    n_idx = n_tok * chr_len
    idx = encodings.reshape(n_idx).astype(jnp.int32)
    out = _build_gather(n_idx, vocab, emb)(table.reshape(vocab * emb), idx)
    mask_out = _build_mask_copy(n_tok, chr_len, mask.dtype.name)(mask)
    return out.reshape(n_tok, chr_len, emb), mask_out
